# baseline (speedup 1.0000x reference)
import functools

import jax
import jax.numpy as jnp
from jax import lax
from jax.experimental import pallas as pl
from jax.experimental.pallas import tpu as pltpu
from jax.experimental.pallas import tpu_sc as plsc

_D = 32
_R = 16384
_K = 20
_B = _R * _K

_info = plsc.get_sparse_core_info()
_NC = _info.num_cores
_NS = _info.num_subcores
_NW = _NC * _NS
_RPW = _R // _NW
_CR = 64
_NCHUNK = _RPW // _CR
_CB = _CR * _K

_mesh = plsc.VectorSubcoreMesh(core_axis_name="c", subcore_axis_name="s")


@functools.partial(
    pl.kernel,
    mesh=_mesh,
    out_type=jax.ShapeDtypeStruct((_B, _D), jnp.float32),
    scratch_types=[
        pltpu.VMEM((_RPW, _K), jnp.int32),
        [pltpu.VMEM((_CB, _D), jnp.float32) for _ in range(2)],
        [pltpu.SemaphoreType.DMA for _ in range(2)],
        [pltpu.SemaphoreType.DMA for _ in range(2)],
    ],
    compiler_params=pltpu.CompilerParams(use_tc_tiling_on_sc=False),
)
def _gather_kernel(table_hbm, idx_hbm, out_hbm, idx_v, obuf, gsem, wsem):
    wid = lax.axis_index("s") * _NC + lax.axis_index("c")
    rbase = wid * _RPW
    obase = wid * _RPW * _K

    pltpu.sync_copy(idx_hbm.at[pl.ds(rbase, _RPW), :], idx_v)

    def issue_chunk(c, b):
        def row(j, carry):
            pltpu.async_copy(
                table_hbm.at[idx_v.at[c * _CR + j, :]],
                obuf[b].at[pl.ds(j * _K, _K), :],
                gsem[b],
            )
            return carry

        lax.fori_loop(0, _CR, row, 0)

    def drain_chunk(b):
        pltpu.make_async_copy(
            table_hbm.at[pl.ds(0, _CB), :], obuf[b], gsem[b]
        ).wait()

    def write_chunk(c, b):
        pltpu.async_copy(
            obuf[b], out_hbm.at[pl.ds(obase + c * _CB, _CB), :], wsem[b]
        )

    def wait_write(c, b):
        pltpu.make_async_copy(
            obuf[b], out_hbm.at[pl.ds(obase + c * _CB, _CB), :], wsem[b]
        ).wait()

    issue_chunk(0, 0)
    for c in range(_NCHUNK):
        b = c % 2
        drain_chunk(b)
        if c + 1 < _NCHUNK:
            if c >= 1:
                wait_write(c - 1, 1 - b)
            issue_chunk(c + 1, 1 - b)
        write_chunk(c, b)
    wait_write(_NCHUNK - 2, _NCHUNK % 2)
    wait_write(_NCHUNK - 1, 1 - _NCHUNK % 2)


def kernel(arg1_1, arg223_1):
    idx = arg223_1.astype(jnp.int32)
    return _gather_kernel(arg1_1, idx)
